# baseline (speedup 1.0000x reference)
import functools

import jax
import jax.numpy as jnp
from jax import lax
from jax.experimental import pallas as pl
from jax.experimental.pallas import tpu as pltpu, tpu_sc as plsc

N = 10000
E = 320000
D = 128
H = 128
C = 6

NPAD = 10240
NWORKERS = 32
CHUNK = 128
CPW = 80
EPAD = NWORKERS * CPW * CHUNK
EROWS = EPAD // CHUNK
ROWS_PER_TILE = NPAD // 16
NBUF = 4


def _drain(z_hbm, dst_ref, sem):
    pltpu.make_async_copy(z_hbm, dst_ref, sem).wait()


def _deg_body(dst2d_hbm, z_hbm, out_hbm, didx, ones, dacc, sem, sem2):
    c = lax.axis_index("c")
    s = lax.axis_index("s")
    wid = s * 2 + c
    rb = pl.multiple_of(s * ROWS_PER_TILE, 8)

    zcp = pltpu.async_copy(z_hbm.at[pl.ds(rb, ROWS_PER_TILE)],
                           dacc.at[pl.ds(rb, ROWS_PER_TILE)], sem)
    icp = pltpu.async_copy(dst2d_hbm.at[pl.ds(wid * CPW, CPW)], didx, sem)

    def fill(r, _):
        ones[r, pl.ds(0, 16)] = jnp.full((16,), 1.0, jnp.float32)
        return 0
    lax.fori_loop(0, CHUNK, fill, 0)

    zcp.wait()
    icp.wait()
    plsc.subcore_barrier()

    W = 16

    def fire(k, _):
        pltpu.async_copy(ones, dacc.at[didx.at[k]], sem2, add=True)

        @pl.when(k >= W)
        def _():
            pltpu.make_async_copy(ones, dacc.at[didx.at[k - W]], sem2).wait()
        return 0
    lax.fori_loop(0, CPW, fire, 0)

    def drain(k, _):
        pltpu.make_async_copy(ones, dacc.at[didx.at[k]], sem2).wait()
        return 0
    lax.fori_loop(CPW - W, CPW, drain, 0)

    plsc.subcore_barrier()
    pltpu.sync_copy(dacc.at[pl.ds(rb, ROWS_PER_TILE)],
                    out_hbm.at[c, pl.ds(rb, ROWS_PER_TILE)])


_deg_kernel = functools.partial(
    pl.kernel,
    mesh=plsc.VectorSubcoreMesh(core_axis_name="c", subcore_axis_name="s"),
    out_type=jax.ShapeDtypeStruct((2, NPAD, 16), jnp.float32),
    scratch_types=[
        pltpu.VMEM((CPW, CHUNK), jnp.int32),
        pltpu.VMEM((CHUNK, 16), jnp.float32),
        pltpu.VMEM_SHARED((NPAD, 16), jnp.float32),
        pltpu.SemaphoreType.DMA,
        pltpu.SemaphoreType.DMA,
    ],
)(_deg_body)


def _scatter_body(src_hbm, dst_hbm, g_hbm, out_hbm,
                  sidx, didx, rows, acc, sem):
    c = lax.axis_index("c")
    s = lax.axis_index("s")
    wid = s * 2 + c
    rb = pl.multiple_of(s * ROWS_PER_TILE, 8)

    def zrow_fill(r, _):
        for j8 in range(D // 16):
            rows[r, pl.ds(j8 * 16, 16)] = jnp.zeros((16,), jnp.float32)
        return 0
    lax.fori_loop(0, CHUNK, zrow_fill, 0)
    for k in range(ROWS_PER_TILE // CHUNK):
        pltpu.sync_copy(rows, acc.at[pl.ds(rb + k * CHUNK, CHUNK)])
    plsc.subcore_barrier()

    ebase = wid * (CPW * CHUNK)

    def body(j, _):
        off = pl.multiple_of(ebase + j * CHUNK, 8)
        pltpu.sync_copy(src_hbm.at[pl.ds(off, CHUNK)], sidx)
        pltpu.sync_copy(dst_hbm.at[pl.ds(off, CHUNK)], didx)
        pltpu.async_copy(g_hbm.at[sidx], rows, sem).wait()
        pltpu.sync_copy(rows, acc.at[didx], add=True)
        return 0

    lax.fori_loop(0, CPW, body, 0)

    plsc.subcore_barrier()
    pltpu.sync_copy(acc.at[pl.ds(rb, ROWS_PER_TILE)],
                    out_hbm.at[c, pl.ds(rb, ROWS_PER_TILE)])


_scatter_kernel = functools.partial(
    pl.kernel,
    mesh=plsc.VectorSubcoreMesh(core_axis_name="c", subcore_axis_name="s"),
    out_type=jax.ShapeDtypeStruct((2, NPAD, D), jnp.float32),
    scratch_types=[
        pltpu.VMEM((CHUNK,), jnp.int32),
        pltpu.VMEM((CHUNK,), jnp.int32),
        pltpu.VMEM((CHUNK, D), jnp.float32),
        pltpu.VMEM_SHARED((NPAD, D), jnp.float32),
        pltpu.SemaphoreType.DMA,
    ],
)(_scatter_body)


def _mm_body(x_ref, w_ref, degp_ref, g_ref, dinv_ref):
    h = jnp.dot(x_ref[...], w_ref[...], preferred_element_type=jnp.float32)
    dsum = degp_ref[0, :N, 0] + degp_ref[1, :N, 0]
    dinv = lax.rsqrt(1.0 + dsum)
    g_ref[...] = h * dinv[:, None]
    dinv_ref[...] = dinv[:, None]


def _ep_body(acc_ref, g_ref, dinv_ref, b1_ref, wfc_ref, bfc_ref, out_ref):
    ssum = acc_ref[0, :N, :] + acc_ref[1, :N, :] + g_ref[...]
    pre = ssum * dinv_ref[...] + b1_ref[...]
    r = jnp.maximum(pre, 0.0)
    out_ref[...] = (jnp.dot(r, wfc_ref[...], preferred_element_type=jnp.float32)
                    + bfc_ref[...])


def kernel(X, edges, W1, b1, Wfc, bfc):
    src = edges[0]
    dst = edges[1]
    npad = EPAD - E
    pad_dst = N + (jnp.arange(npad, dtype=dst.dtype) % (NPAD - N))
    src_p = jnp.concatenate([src, jnp.zeros((npad,), src.dtype)])
    dst_p = jnp.concatenate([dst, pad_dst])
    dst2d = dst_p.reshape(EROWS, CHUNK)
    zdeg = jnp.zeros((NPAD, 16), jnp.float32)

    degp = _deg_kernel(dst2d, zdeg)

    g, dinv = pl.pallas_call(
        _mm_body,
        out_shape=[
            jax.ShapeDtypeStruct((N, H), jnp.float32),
            jax.ShapeDtypeStruct((N, 1), jnp.float32),
        ],
    )(X, W1, degp)

    acc = _scatter_kernel(src_p, dst_p, g)

    out = pl.pallas_call(
        _ep_body,
        out_shape=jax.ShapeDtypeStruct((N, C), jnp.float32),
    )(acc, g, dinv, b1.reshape(1, H), Wfc, bfc.reshape(1, C))
    return out


# speedup vs baseline: 1.5634x; 1.5634x over previous
import functools

import jax
import jax.numpy as jnp
from jax import lax
from jax.experimental import pallas as pl
from jax.experimental.pallas import tpu as pltpu, tpu_sc as plsc

N = 10000
E = 320000
D = 128
H = 128
C = 6

NPAD = 10240
NWORKERS = 32
CHUNK = 128
CPW = 80
EPAD = NWORKERS * CPW * CHUNK
EROWS = EPAD // CHUNK
ROWS_PER_TILE = NPAD // 16
NBUF = 4


def _drain(z_hbm, dst_ref, sem):
    pltpu.make_async_copy(z_hbm, dst_ref, sem).wait()


def _deg_body(dst2d_hbm, z_hbm, out_hbm, didx, ones, dacc, sem, sem2):
    c = lax.axis_index("c")
    s = lax.axis_index("s")
    wid = s * 2 + c
    rb = pl.multiple_of(s * ROWS_PER_TILE, 8)

    zcp = pltpu.async_copy(z_hbm.at[pl.ds(rb, ROWS_PER_TILE)],
                           dacc.at[pl.ds(rb, ROWS_PER_TILE)], sem)
    icp = pltpu.async_copy(dst2d_hbm.at[pl.ds(wid * CPW, CPW)], didx, sem)

    def fill(r, _):
        ones[r, pl.ds(0, 16)] = jnp.full((16,), 1.0, jnp.float32)
        return 0
    lax.fori_loop(0, CHUNK, fill, 0)

    zcp.wait()
    icp.wait()
    plsc.subcore_barrier()

    def fire(k, _):
        pltpu.async_copy(ones, dacc.at[didx.at[k]], sem2, add=True).wait()
        return 0
    lax.fori_loop(0, CPW, fire, 0)

    plsc.subcore_barrier()
    pltpu.sync_copy(dacc.at[pl.ds(rb, ROWS_PER_TILE)],
                    out_hbm.at[c, pl.ds(rb, ROWS_PER_TILE)])


_deg_kernel = functools.partial(
    pl.kernel,
    mesh=plsc.VectorSubcoreMesh(core_axis_name="c", subcore_axis_name="s"),
    out_type=jax.ShapeDtypeStruct((2, NPAD, 16), jnp.float32),
    scratch_types=[
        pltpu.VMEM((CPW, CHUNK), jnp.int32),
        pltpu.VMEM((CHUNK, 16), jnp.float32),
        pltpu.VMEM_SHARED((NPAD, 16), jnp.float32),
        pltpu.SemaphoreType.DMA,
        pltpu.SemaphoreType.DMA,
    ],
)(_deg_body)


HCPW = CPW // 2


def _scatter_body(src2d_hbm, dst2d_hbm, g_hbm, out_hbm,
                  sidx, didx, rows0, rows1, acc, si, sg0, sg1):
    c = lax.axis_index("c")
    s = lax.axis_index("s")
    wid = s * 2 + c
    rb = pl.multiple_of(s * ROWS_PER_TILE, 8)
    rows = [rows0, rows1]
    sg = [sg0, sg1]
    erow0 = wid * CPW

    scp = pltpu.async_copy(src2d_hbm.at[pl.ds(erow0, HCPW)], sidx, si)
    dcp = pltpu.async_copy(dst2d_hbm.at[pl.ds(erow0, HCPW)], didx, si)

    def zrow_fill(r, _):
        for j8 in range(D // 16):
            rows1[r, pl.ds(j8 * 16, 16)] = jnp.zeros((16,), jnp.float32)
        return 0
    lax.fori_loop(0, CHUNK, zrow_fill, 0)
    for k in range(ROWS_PER_TILE // CHUNK):
        pltpu.sync_copy(rows1, acc.at[pl.ds(rb + k * CHUNK, CHUNK)])
    scp.wait()
    dcp.wait()
    pltpu.async_copy(g_hbm.at[sidx.at[0]], rows0, sg0)
    plsc.subcore_barrier()

    def phase(p):
        def grp(gi, _):
            for b in range(2):
                k = gi * 2 + b
                bo = 1 - b
                pltpu.make_async_copy(
                    g_hbm.at[sidx.at[k]], rows[b], sg[b]).wait()

                @pl.when(k + 1 < HCPW)
                def _():
                    pltpu.async_copy(
                        g_hbm.at[sidx.at[k + 1]], rows[bo], sg[bo])

                pltpu.sync_copy(rows[b], acc.at[didx.at[k]], add=True)
            return 0

        lax.fori_loop(0, HCPW // 2, grp, 0)

    phase(0)
    pltpu.async_copy(
        src2d_hbm.at[pl.ds(erow0 + HCPW, HCPW)], sidx, si).wait()
    pltpu.async_copy(
        dst2d_hbm.at[pl.ds(erow0 + HCPW, HCPW)], didx, si).wait()
    pltpu.async_copy(g_hbm.at[sidx.at[0]], rows0, sg0)
    phase(1)

    plsc.subcore_barrier()
    pltpu.sync_copy(acc.at[pl.ds(rb, ROWS_PER_TILE)],
                    out_hbm.at[c, pl.ds(rb, ROWS_PER_TILE)])


_scatter_kernel = functools.partial(
    pl.kernel,
    mesh=plsc.VectorSubcoreMesh(core_axis_name="c", subcore_axis_name="s"),
    out_type=jax.ShapeDtypeStruct((2, NPAD, D), jnp.float32),
    scratch_types=[
        pltpu.VMEM((HCPW, CHUNK), jnp.int32),
        pltpu.VMEM((HCPW, CHUNK), jnp.int32),
        pltpu.VMEM((CHUNK, D), jnp.float32),
        pltpu.VMEM((CHUNK, D), jnp.float32),
        pltpu.VMEM_SHARED((NPAD, D), jnp.float32),
        pltpu.SemaphoreType.DMA,
        pltpu.SemaphoreType.DMA,
        pltpu.SemaphoreType.DMA,
    ],
)(_scatter_body)


def _mm_body(x_ref, w_ref, degp_ref, g_ref, dinv_ref):
    h = jnp.dot(x_ref[...], w_ref[...], preferred_element_type=jnp.float32)
    dsum = degp_ref[0, :N, 0] + degp_ref[1, :N, 0]
    dinv = lax.rsqrt(1.0 + dsum)
    g_ref[...] = h * dinv[:, None]
    dinv_ref[...] = dinv[:, None]


def _ep_body(acc_ref, g_ref, dinv_ref, b1_ref, wfc_ref, bfc_ref, out_ref):
    ssum = acc_ref[0, :N, :] + acc_ref[1, :N, :] + g_ref[...]
    pre = ssum * dinv_ref[...] + b1_ref[...]
    r = jnp.maximum(pre, 0.0)
    out_ref[...] = (jnp.dot(r, wfc_ref[...], preferred_element_type=jnp.float32)
                    + bfc_ref[...])


def kernel(X, edges, W1, b1, Wfc, bfc):
    src = edges[0]
    dst = edges[1]
    epw = E // NWORKERS
    ppw = CPW * CHUNK - epw
    pad_src = jnp.zeros((NWORKERS, ppw), src.dtype)
    pad_dst = jnp.broadcast_to(
        N + jnp.arange(NWORKERS, dtype=dst.dtype)[:, None], (NWORKERS, ppw))
    src2d = jnp.concatenate(
        [src.reshape(NWORKERS, epw), pad_src], axis=1).reshape(EROWS, CHUNK)
    dst2d = jnp.concatenate(
        [dst.reshape(NWORKERS, epw), pad_dst], axis=1).reshape(EROWS, CHUNK)
    zdeg = jnp.zeros((NPAD, 16), jnp.float32)

    degp = _deg_kernel(dst2d, zdeg)

    g, dinv = pl.pallas_call(
        _mm_body,
        out_shape=[
            jax.ShapeDtypeStruct((N, H), jnp.float32),
            jax.ShapeDtypeStruct((N, 1), jnp.float32),
        ],
    )(X, W1, degp)

    acc = _scatter_kernel(src2d, dst2d, g)

    out = pl.pallas_call(
        _ep_body,
        out_shape=jax.ShapeDtypeStruct((N, C), jnp.float32),
    )(acc, g, dinv, b1.reshape(1, H), Wfc, bfc.reshape(1, C))
    return out


# speedup vs baseline: 1.5676x; 1.0027x over previous
import functools

import jax
import jax.numpy as jnp
from jax import lax
from jax.experimental import pallas as pl
from jax.experimental.pallas import tpu as pltpu, tpu_sc as plsc

N = 10000
E = 320000
D = 128
H = 128
C = 6

NPAD = 10240
NWORKERS = 32
CHUNK = 128
CPW = 80
EPAD = NWORKERS * CPW * CHUNK
EROWS = EPAD // CHUNK
ROWS_PER_TILE = NPAD // 16
NBUF = 4


def _drain(z_hbm, dst_ref, sem):
    pltpu.make_async_copy(z_hbm, dst_ref, sem).wait()


def _deg_body(dst2d_hbm, z_hbm, out_hbm, didx, ones, dacc, sem, sem2):
    c = lax.axis_index("c")
    s = lax.axis_index("s")
    wid = s * 2 + c
    rb = pl.multiple_of(s * ROWS_PER_TILE, 8)

    zcp = pltpu.async_copy(z_hbm.at[pl.ds(rb, ROWS_PER_TILE)],
                           dacc.at[pl.ds(rb, ROWS_PER_TILE)], sem)
    icp = pltpu.async_copy(dst2d_hbm.at[pl.ds(wid * CPW, CPW)], didx, sem)

    def fill(r, _):
        ones[r, pl.ds(0, 16)] = jnp.full((16,), 1.0, jnp.float32)
        return 0
    lax.fori_loop(0, CHUNK, fill, 0)

    zcp.wait()
    icp.wait()
    plsc.subcore_barrier()

    def fire(k, _):
        pltpu.async_copy(ones, dacc.at[didx.at[k]], sem2, add=True).wait()
        return 0
    lax.fori_loop(0, CPW, fire, 0)

    plsc.subcore_barrier()
    pltpu.sync_copy(dacc.at[pl.ds(rb, ROWS_PER_TILE)],
                    out_hbm.at[c, pl.ds(rb, ROWS_PER_TILE)])


_deg_kernel = functools.partial(
    pl.kernel,
    mesh=plsc.VectorSubcoreMesh(core_axis_name="c", subcore_axis_name="s"),
    out_type=jax.ShapeDtypeStruct((2, NPAD, 16), jnp.float32),
    scratch_types=[
        pltpu.VMEM((CPW, CHUNK), jnp.int32),
        pltpu.VMEM((CHUNK, 16), jnp.float32),
        pltpu.VMEM_SHARED((NPAD, 16), jnp.float32),
        pltpu.SemaphoreType.DMA,
        pltpu.SemaphoreType.DMA,
    ],
)(_deg_body)


def _scatter_body(src2d_hbm, dst2d_hbm, g_hbm, out_hbm,
                  sidx0, sidx1, didx0, didx1, rows0, rows1, acc,
                  si0, si1, sg0, sg1):
    c = lax.axis_index("c")
    s = lax.axis_index("s")
    wid = s * 2 + c
    rb = pl.multiple_of(s * ROWS_PER_TILE, 8)
    rows = [rows0, rows1]
    sidx = [sidx0, sidx1]
    didx = [didx0, didx1]
    si = [si0, si1]
    sg = [sg0, sg1]
    erow0 = wid * CPW

    for j in range(2):
        pltpu.async_copy(src2d_hbm.at[erow0 + j], sidx[j], si[j])
        pltpu.async_copy(dst2d_hbm.at[erow0 + j], didx[j], si[j])

    def zrow_fill(r, _):
        for j8 in range(D // 16):
            rows1[r, pl.ds(j8 * 16, 16)] = jnp.zeros((16,), jnp.float32)
        return 0
    lax.fori_loop(0, CHUNK, zrow_fill, 0)
    for k in range(ROWS_PER_TILE // CHUNK):
        pltpu.sync_copy(rows1, acc.at[pl.ds(rb + k * CHUNK, CHUNK)])

    _drain(src2d_hbm.at[erow0], sidx0, si0)
    _drain(dst2d_hbm.at[erow0], didx0, si0)
    pltpu.async_copy(g_hbm.at[sidx0], rows0, sg0)
    plsc.subcore_barrier()

    def grp(gi, _):
        for b in range(2):
            k = gi * 2 + b
            bo = 1 - b
            pltpu.make_async_copy(g_hbm.at[sidx[b]], rows[b], sg[b]).wait()

            @pl.when(k + 1 < CPW)
            def _():
                _drain(src2d_hbm.at[erow0], sidx[bo], si[bo])
                _drain(dst2d_hbm.at[erow0], didx[bo], si[bo])
                pltpu.async_copy(g_hbm.at[sidx[bo]], rows[bo], sg[bo])

            pltpu.sync_copy(rows[b], acc.at[didx[b]], add=True)

            @pl.when(k + 2 < CPW)
            def _():
                pltpu.async_copy(src2d_hbm.at[erow0 + k + 2], sidx[b], si[b])
                pltpu.async_copy(dst2d_hbm.at[erow0 + k + 2], didx[b], si[b])
        return 0

    lax.fori_loop(0, CPW // 2, grp, 0)

    plsc.subcore_barrier()
    pltpu.sync_copy(acc.at[pl.ds(rb, ROWS_PER_TILE)],
                    out_hbm.at[c, pl.ds(rb, ROWS_PER_TILE)])


_scatter_kernel = functools.partial(
    pl.kernel,
    mesh=plsc.VectorSubcoreMesh(core_axis_name="c", subcore_axis_name="s"),
    out_type=jax.ShapeDtypeStruct((2, NPAD, D), jnp.float32),
    scratch_types=[
        pltpu.VMEM((CHUNK,), jnp.int32),
        pltpu.VMEM((CHUNK,), jnp.int32),
        pltpu.VMEM((CHUNK,), jnp.int32),
        pltpu.VMEM((CHUNK,), jnp.int32),
        pltpu.VMEM((CHUNK, D), jnp.float32),
        pltpu.VMEM((CHUNK, D), jnp.float32),
        pltpu.VMEM_SHARED((NPAD, D), jnp.float32),
        pltpu.SemaphoreType.DMA,
        pltpu.SemaphoreType.DMA,
        pltpu.SemaphoreType.DMA,
        pltpu.SemaphoreType.DMA,
    ],
)(_scatter_body)


def _mm_body(x_ref, w_ref, degp_ref, g_ref, dinv_ref):
    h = jnp.dot(x_ref[...], w_ref[...], preferred_element_type=jnp.float32)
    dsum = degp_ref[0, :N, 0] + degp_ref[1, :N, 0]
    dinv = lax.rsqrt(1.0 + dsum)
    g_ref[...] = h * dinv[:, None]
    dinv_ref[...] = dinv[:, None]


def _ep_body(acc_ref, g_ref, dinv_ref, b1_ref, wfc_ref, bfc_ref, out_ref):
    ssum = acc_ref[0, :N, :] + acc_ref[1, :N, :] + g_ref[...]
    pre = ssum * dinv_ref[...] + b1_ref[...]
    r = jnp.maximum(pre, 0.0)
    out_ref[...] = (jnp.dot(r, wfc_ref[...], preferred_element_type=jnp.float32)
                    + bfc_ref[...])


def kernel(X, edges, W1, b1, Wfc, bfc):
    src = edges[0]
    dst = edges[1]
    epw = E // NWORKERS
    ppw = CPW * CHUNK - epw
    pad_src = jnp.zeros((NWORKERS, ppw), src.dtype)
    pad_dst = jnp.broadcast_to(
        N + jnp.arange(NWORKERS, dtype=dst.dtype)[:, None], (NWORKERS, ppw))
    src2d = jnp.concatenate(
        [src.reshape(NWORKERS, epw), pad_src], axis=1).reshape(EROWS, CHUNK)
    dst2d = jnp.concatenate(
        [dst.reshape(NWORKERS, epw), pad_dst], axis=1).reshape(EROWS, CHUNK)
    zdeg = jnp.zeros((NPAD, 16), jnp.float32)

    degp = _deg_kernel(dst2d, zdeg)

    g, dinv = pl.pallas_call(
        _mm_body,
        out_shape=[
            jax.ShapeDtypeStruct((N, H), jnp.float32),
            jax.ShapeDtypeStruct((N, 1), jnp.float32),
        ],
    )(X, W1, degp)

    acc = _scatter_kernel(src2d, dst2d, g)

    out = pl.pallas_call(
        _ep_body,
        out_shape=jax.ShapeDtypeStruct((N, C), jnp.float32),
    )(acc, g, dinv, b1.reshape(1, H), Wfc, bfc.reshape(1, C))
    return out
